# Initial kernel scaffold; baseline (speedup 1.0000x reference)
import jax
import jax.numpy as jnp
from jax.experimental import pallas as pl


def kernel(inputs, pos_table):
    raise NotImplementedError("write your pallas kernel here")


# speedup vs baseline: 1.4470x; 1.4470x over previous
import jax
import jax.numpy as jnp
from jax.experimental import pallas as pl


def _add_block(x_ref, p_ref, o_ref):
    o_ref[...] = x_ref[...] + p_ref[...]


def kernel(inputs, pos_table):
    B, L, D = inputs.shape
    BL = 512

    return pl.pallas_call(
        _add_block,
        grid=(L // BL, B),
        in_specs=[
            pl.BlockSpec((1, BL, D), lambda l, b: (b, l, 0)),
            pl.BlockSpec((BL, D), lambda l, b: (l, 0)),
        ],
        out_specs=pl.BlockSpec((1, BL, D), lambda l, b: (b, l, 0)),
        out_shape=jax.ShapeDtypeStruct((B, L, D), inputs.dtype),
    )(inputs, pos_table)


# speedup vs baseline: 1.6761x; 1.1584x over previous
import jax
import jax.numpy as jnp
from jax.experimental import pallas as pl


def _add_block(x_ref, p_ref, o_ref):
    o_ref[...] = x_ref[...] + p_ref[...]


def kernel(inputs, pos_table):
    B, L, D = inputs.shape
    BL = 1024

    return pl.pallas_call(
        _add_block,
        grid=(L // BL, B),
        in_specs=[
            pl.BlockSpec((1, BL, D), lambda l, b: (b, l, 0)),
            pl.BlockSpec((BL, D), lambda l, b: (l, 0)),
        ],
        out_specs=pl.BlockSpec((1, BL, D), lambda l, b: (b, l, 0)),
        out_shape=jax.ShapeDtypeStruct((B, L, D), inputs.dtype),
    )(inputs, pos_table)


# speedup vs baseline: 1.7961x; 1.0716x over previous
import jax
import jax.numpy as jnp
from jax.experimental import pallas as pl


def _add_block(x_ref, p_ref, o_ref):
    o_ref[...] = x_ref[...] + p_ref[...]


def kernel(inputs, pos_table):
    B, L, D = inputs.shape
    BL = 2048

    return pl.pallas_call(
        _add_block,
        grid=(L // BL, B),
        in_specs=[
            pl.BlockSpec((1, BL, D), lambda l, b: (b, l, 0)),
            pl.BlockSpec((BL, D), lambda l, b: (l, 0)),
        ],
        out_specs=pl.BlockSpec((1, BL, D), lambda l, b: (b, l, 0)),
        out_shape=jax.ShapeDtypeStruct((B, L, D), inputs.dtype),
    )(inputs, pos_table)


# speedup vs baseline: 1.7978x; 1.0010x over previous
import jax
import jax.numpy as jnp
from jax.experimental import pallas as pl


def _add_block(x_ref, p_ref, o_ref):
    o_ref[...] = x_ref[...] + p_ref[...]


def kernel(inputs, pos_table):
    B, L, D = inputs.shape
    BL = 1024

    return pl.pallas_call(
        _add_block,
        grid=(L // BL,),
        in_specs=[
            pl.BlockSpec((B, BL, D), lambda l: (0, l, 0)),
            pl.BlockSpec((BL, D), lambda l: (l, 0)),
        ],
        out_specs=pl.BlockSpec((B, BL, D), lambda l: (0, l, 0)),
        out_shape=jax.ShapeDtypeStruct((B, L, D), inputs.dtype),
    )(inputs, pos_table)
